# baseline (speedup 1.0000x reference)
import jax
import jax.numpy as jnp
from jax.experimental import pallas as pl
from jax.experimental.pallas import tpu as pltpu

_G = 256
_M = 32
_BIG = 3.0e38


def _fps_kernel(xyz_ref, idx_ref, cen_ref, md_ref):
    _, B, N = xyz_ref.shape
    x = xyz_ref[0]
    y = xyz_ref[1]
    z = xyz_ref[2]
    iota = jax.lax.broadcasted_iota(jnp.int32, (B, N), 1)
    slot = jax.lax.broadcasted_iota(jnp.int32, (B, _G), 1)

    cx0 = x[:, 0:1]
    cy0 = y[:, 0:1]
    cz0 = z[:, 0:1]
    dx = x - cx0
    dy = y - cy0
    dz = z - cz0
    md_ref[...] = dx * dx + dy * dy + dz * dz

    zero_g = jnp.zeros((B, _G), jnp.float32)
    ax0 = jnp.where(slot == 0, jnp.broadcast_to(cx0, (B, _G)), zero_g)
    ay0 = jnp.where(slot == 0, jnp.broadcast_to(cy0, (B, _G)), zero_g)
    az0 = jnp.where(slot == 0, jnp.broadcast_to(cz0, (B, _G)), zero_g)

    def step(t, carry):
        idx_acc, ax, ay, az = carry
        md = md_ref[...]
        m = jnp.max(md, axis=1, keepdims=True)
        nxt = jnp.min(jnp.where(md == m, iota, N), axis=1, keepdims=True)
        idx_acc = jnp.where(slot == t, jnp.broadcast_to(nxt, (B, _G)), idx_acc)
        sel = iota == nxt
        zero = jnp.zeros_like(x)
        cx = jnp.sum(jnp.where(sel, x, zero), axis=1, keepdims=True)
        cy = jnp.sum(jnp.where(sel, y, zero), axis=1, keepdims=True)
        cz = jnp.sum(jnp.where(sel, z, zero), axis=1, keepdims=True)
        ax = jnp.where(slot == t, jnp.broadcast_to(cx, (B, _G)), ax)
        ay = jnp.where(slot == t, jnp.broadcast_to(cy, (B, _G)), ay)
        az = jnp.where(slot == t, jnp.broadcast_to(cz, (B, _G)), az)
        dx = x - cx
        dy = y - cy
        dz = z - cz
        d = dx * dx + dy * dy + dz * dz
        md_ref[...] = jnp.minimum(md, d)
        return idx_acc, ax, ay, az

    idx0 = jnp.zeros((B, _G), jnp.int32)
    idx, ax, ay, az = jax.lax.fori_loop(1, _G, step, (idx0, ax0, ay0, az0))
    idx_ref[...] = idx
    cen_ref[0] = ax
    cen_ref[1] = ay
    cen_ref[2] = az


def _knn_kernel(xyz_ref, cen_ref, nb_ref, md_ref):
    _, _, N = xyz_ref.shape
    a = xyz_ref[0]
    x = a[0:1]
    y = a[1:2]
    z = a[2:3]
    cen = cen_ref[0]
    cx = cen[:, 0:1]
    cy = cen[:, 1:2]
    cz = cen[:, 2:3]

    x2 = x * x + y * y + z * z
    c2 = cx * cx + cy * cy + cz * cz
    bx = jnp.broadcast_to(x, (_G, N))
    by = jnp.broadcast_to(y, (_G, N))
    bz = jnp.broadcast_to(z, (_G, N))
    cross = jax.lax.dot_general(
        cen, a, (((1,), (0,)), ((), ())),
        preferred_element_type=jnp.float32)
    md_ref[...] = (c2 + x2) - 2.0 * cross

    iota = jax.lax.broadcasted_iota(jnp.int32, (_G, N), 1)
    slotm = jax.lax.broadcasted_iota(jnp.int32, (_G, _M), 1)
    zero = jnp.zeros((_G, N), jnp.float32)

    def step(t, carry):
        nbx, nby, nbz = carry
        md = md_ref[...]
        m = jnp.min(md, axis=1, keepdims=True)
        nxt = jnp.min(jnp.where(md == m, iota, N), axis=1, keepdims=True)
        sel = iota == nxt
        px = jnp.sum(jnp.where(sel, bx, zero), axis=1, keepdims=True)
        py = jnp.sum(jnp.where(sel, by, zero), axis=1, keepdims=True)
        pz = jnp.sum(jnp.where(sel, bz, zero), axis=1, keepdims=True)
        nbx = jnp.where(slotm == t, jnp.broadcast_to(px, (_G, _M)), nbx)
        nby = jnp.where(slotm == t, jnp.broadcast_to(py, (_G, _M)), nby)
        nbz = jnp.where(slotm == t, jnp.broadcast_to(pz, (_G, _M)), nbz)
        md_ref[...] = jnp.where(sel, _BIG, md)
        return nbx, nby, nbz

    nb0 = jnp.zeros((_G, _M), jnp.float32)
    nbx, nby, nbz = jax.lax.fori_loop(0, _M, step, (nb0, nb0, nb0))
    nb_ref[0, 0] = nbx - cx
    nb_ref[0, 1] = nby - cy
    nb_ref[0, 2] = nbz - cz


def kernel(xyz):
    B, N, _ = xyz.shape
    xyz_t = jnp.transpose(xyz, (2, 0, 1))
    fps_idx, cen = pl.pallas_call(
        _fps_kernel,
        out_shape=(
            jax.ShapeDtypeStruct((B, _G), jnp.int32),
            jax.ShapeDtypeStruct((3, B, _G), jnp.float32),
        ),
        scratch_shapes=[pltpu.VMEM((B, N), jnp.float32)],
    )(xyz_t)

    center = jnp.transpose(cen, (1, 2, 0))
    xyz_b = jnp.transpose(xyz, (0, 2, 1))
    nb = pl.pallas_call(
        _knn_kernel,
        grid=(B,),
        in_specs=[
            pl.BlockSpec((1, 3, N), lambda b: (b, 0, 0)),
            pl.BlockSpec((1, _G, 3), lambda b: (b, 0, 0)),
        ],
        out_specs=pl.BlockSpec((1, 3, _G, _M), lambda b: (b, 0, 0, 0)),
        out_shape=jax.ShapeDtypeStruct((B, 3, _G, _M), jnp.float32),
        scratch_shapes=[pltpu.VMEM((_G, N), jnp.float32)],
        compiler_params=pltpu.CompilerParams(
            dimension_semantics=("parallel",)),
    )(xyz_b, center)

    neighborhood = jnp.transpose(nb, (0, 2, 3, 1))
    return neighborhood, center, fps_idx


# speedup vs baseline: 1.6005x; 1.6005x over previous
import jax
import jax.numpy as jnp
from jax.experimental import pallas as pl
from jax.experimental.pallas import tpu as pltpu

_G = 256
_M = 32
_BIG = 3.0e38


def _fps_kernel(xyz_ref, idx_ref, cen_ref, md_ref):
    _, B, N = xyz_ref.shape
    x = xyz_ref[0]
    y = xyz_ref[1]
    z = xyz_ref[2]
    iota = jax.lax.broadcasted_iota(jnp.int32, (B, N), 1)
    slot = jax.lax.broadcasted_iota(jnp.int32, (B, _G), 1)

    cx0 = x[:, 0:1]
    cy0 = y[:, 0:1]
    cz0 = z[:, 0:1]
    dx = x - cx0
    dy = y - cy0
    dz = z - cz0
    md_ref[...] = dx * dx + dy * dy + dz * dz

    zero_g = jnp.zeros((B, _G), jnp.float32)
    ax0 = jnp.where(slot == 0, jnp.broadcast_to(cx0, (B, _G)), zero_g)
    ay0 = jnp.where(slot == 0, jnp.broadcast_to(cy0, (B, _G)), zero_g)
    az0 = jnp.where(slot == 0, jnp.broadcast_to(cz0, (B, _G)), zero_g)

    def step(t, carry):
        idx_acc, ax, ay, az = carry
        md = md_ref[...]
        m = jnp.max(md, axis=1, keepdims=True)
        nxt = jnp.min(jnp.where(md == m, iota, N), axis=1, keepdims=True)
        idx_acc = jnp.where(slot == t, jnp.broadcast_to(nxt, (B, _G)), idx_acc)
        sel = iota == nxt
        zero = jnp.zeros_like(x)
        cx = jnp.sum(jnp.where(sel, x, zero), axis=1, keepdims=True)
        cy = jnp.sum(jnp.where(sel, y, zero), axis=1, keepdims=True)
        cz = jnp.sum(jnp.where(sel, z, zero), axis=1, keepdims=True)
        ax = jnp.where(slot == t, jnp.broadcast_to(cx, (B, _G)), ax)
        ay = jnp.where(slot == t, jnp.broadcast_to(cy, (B, _G)), ay)
        az = jnp.where(slot == t, jnp.broadcast_to(cz, (B, _G)), az)
        dx = x - cx
        dy = y - cy
        dz = z - cz
        d = dx * dx + dy * dy + dz * dz
        md_ref[...] = jnp.minimum(md, d)
        return idx_acc, ax, ay, az

    idx0 = jnp.zeros((B, _G), jnp.int32)
    idx, ax, ay, az = jax.lax.fori_loop(1, _G, step, (idx0, ax0, ay0, az0))
    idx_ref[...] = idx
    cen_ref[0] = ax
    cen_ref[1] = ay
    cen_ref[2] = az


def _knn_kernel(xyz_ref, xyzn_ref, cen_ref, nb_ref, md_ref):
    _, _, N = xyz_ref.shape
    a = xyz_ref[0]
    x = a[0:1]
    y = a[1:2]
    z = a[2:3]
    xyzn = xyzn_ref[0]
    cen = cen_ref[0]
    cx = cen[:, 0:1]
    cy = cen[:, 1:2]
    cz = cen[:, 2:3]

    x2 = x * x + y * y + z * z
    c2 = cx * cx + cy * cy + cz * cz
    cross = jax.lax.dot_general(
        cen, a, (((1,), (0,)), ((), ())),
        preferred_element_type=jnp.float32)
    md_ref[...] = (c2 + x2) - 2.0 * cross

    fiota = jax.lax.broadcasted_iota(jnp.int32, (_G, N), 1).astype(jnp.float32)
    slotm = jax.lax.broadcasted_iota(jnp.int32, (_G, _M), 1)
    fN = float(N)

    def step(t, carry):
        nbx, nby, nbz = carry
        md = md_ref[...]
        m = jnp.min(md, axis=1, keepdims=True)
        nxt = jnp.min(jnp.where(md == m, fiota, fN), axis=1, keepdims=True)
        sel = fiota == nxt
        md_ref[...] = jnp.where(sel, _BIG, md)
        self32 = sel.astype(jnp.float32)
        p = jax.lax.dot_general(
            self32, xyzn, (((1,), (0,)), ((), ())),
            preferred_element_type=jnp.float32)
        nbx = jnp.where(slotm == t, jnp.broadcast_to(p[:, 0:1], (_G, _M)), nbx)
        nby = jnp.where(slotm == t, jnp.broadcast_to(p[:, 1:2], (_G, _M)), nby)
        nbz = jnp.where(slotm == t, jnp.broadcast_to(p[:, 2:3], (_G, _M)), nbz)
        return nbx, nby, nbz

    nb0 = jnp.zeros((_G, _M), jnp.float32)
    nbx, nby, nbz = jax.lax.fori_loop(0, _M, step, (nb0, nb0, nb0))
    nb_ref[0, 0] = nbx - cx
    nb_ref[0, 1] = nby - cy
    nb_ref[0, 2] = nbz - cz


def kernel(xyz):
    B, N, _ = xyz.shape
    xyz_t = jnp.transpose(xyz, (2, 0, 1))
    fps_idx, cen = pl.pallas_call(
        _fps_kernel,
        out_shape=(
            jax.ShapeDtypeStruct((B, _G), jnp.int32),
            jax.ShapeDtypeStruct((3, B, _G), jnp.float32),
        ),
        scratch_shapes=[pltpu.VMEM((B, N), jnp.float32)],
    )(xyz_t)

    center = jnp.transpose(cen, (1, 2, 0))
    xyz_b = jnp.transpose(xyz, (0, 2, 1))
    nb = pl.pallas_call(
        _knn_kernel,
        grid=(B,),
        in_specs=[
            pl.BlockSpec((1, 3, N), lambda b: (b, 0, 0)),
            pl.BlockSpec((1, N, 3), lambda b: (b, 0, 0)),
            pl.BlockSpec((1, _G, 3), lambda b: (b, 0, 0)),
        ],
        out_specs=pl.BlockSpec((1, 3, _G, _M), lambda b: (b, 0, 0, 0)),
        out_shape=jax.ShapeDtypeStruct((B, 3, _G, _M), jnp.float32),
        scratch_shapes=[pltpu.VMEM((_G, N), jnp.float32)],
        compiler_params=pltpu.CompilerParams(
            dimension_semantics=("parallel",)),
    )(xyz_b, xyz, center)

    neighborhood = jnp.transpose(nb, (0, 2, 3, 1))
    return neighborhood, center, fps_idx


# speedup vs baseline: 1.6195x; 1.0119x over previous
import jax
import jax.numpy as jnp
from jax.experimental import pallas as pl
from jax.experimental.pallas import tpu as pltpu

_G = 256
_M = 32
_BIG = 3.0e38


def _fps_kernel(xyz_ref, idx_ref, cen_ref, md_ref):
    _, B, N = xyz_ref.shape
    x = xyz_ref[0]
    y = xyz_ref[1]
    z = xyz_ref[2]
    iota = jax.lax.broadcasted_iota(jnp.int32, (B, N), 1)
    slot = jax.lax.broadcasted_iota(jnp.int32, (B, _G), 1)

    cx0 = x[:, 0:1]
    cy0 = y[:, 0:1]
    cz0 = z[:, 0:1]
    dx = x - cx0
    dy = y - cy0
    dz = z - cz0
    md_ref[...] = dx * dx + dy * dy + dz * dz

    zero_g = jnp.zeros((B, _G), jnp.float32)
    ax0 = jnp.where(slot == 0, jnp.broadcast_to(cx0, (B, _G)), zero_g)
    ay0 = jnp.where(slot == 0, jnp.broadcast_to(cy0, (B, _G)), zero_g)
    az0 = jnp.where(slot == 0, jnp.broadcast_to(cz0, (B, _G)), zero_g)

    def step(t, carry):
        idx_acc, ax, ay, az = carry
        md = md_ref[...]
        nxt = jnp.argmax(md, axis=1)[:, None]
        idx_acc = jnp.where(slot == t, jnp.broadcast_to(nxt, (B, _G)), idx_acc)
        sel = iota == nxt
        zero = jnp.zeros_like(x)
        cx = jnp.sum(jnp.where(sel, x, zero), axis=1, keepdims=True)
        cy = jnp.sum(jnp.where(sel, y, zero), axis=1, keepdims=True)
        cz = jnp.sum(jnp.where(sel, z, zero), axis=1, keepdims=True)
        ax = jnp.where(slot == t, jnp.broadcast_to(cx, (B, _G)), ax)
        ay = jnp.where(slot == t, jnp.broadcast_to(cy, (B, _G)), ay)
        az = jnp.where(slot == t, jnp.broadcast_to(cz, (B, _G)), az)
        dx = x - cx
        dy = y - cy
        dz = z - cz
        d = dx * dx + dy * dy + dz * dz
        md_ref[...] = jnp.minimum(md, d)
        return idx_acc, ax, ay, az

    idx0 = jnp.zeros((B, _G), jnp.int32)
    idx, ax, ay, az = jax.lax.fori_loop(1, _G, step, (idx0, ax0, ay0, az0))
    idx_ref[...] = idx
    cen_ref[0] = ax
    cen_ref[1] = ay
    cen_ref[2] = az


def _knn_kernel(xyz_ref, xyzn_ref, cen_ref, nb_ref, md_ref):
    _, _, N = xyz_ref.shape
    a = xyz_ref[0]
    x = a[0:1]
    y = a[1:2]
    z = a[2:3]
    xyzn = xyzn_ref[0]
    cen = cen_ref[0]
    cx = cen[:, 0:1]
    cy = cen[:, 1:2]
    cz = cen[:, 2:3]

    x2 = x * x + y * y + z * z
    c2 = cx * cx + cy * cy + cz * cz
    cross = jax.lax.dot_general(
        cen, a, (((1,), (0,)), ((), ())),
        preferred_element_type=jnp.float32)
    md_ref[...] = (c2 + x2) - 2.0 * cross

    fiota = jax.lax.broadcasted_iota(jnp.int32, (_G, N), 1).astype(jnp.float32)
    slotm = jax.lax.broadcasted_iota(jnp.int32, (_G, _M), 1)
    fN = float(N)

    def step(t, carry):
        nbx, nby, nbz = carry
        md = md_ref[...]
        m = jnp.min(md, axis=1, keepdims=True)
        nxt = jnp.min(jnp.where(md == m, fiota, fN), axis=1, keepdims=True)
        sel = fiota == nxt
        md_ref[...] = jnp.where(sel, _BIG, md)
        self32 = sel.astype(jnp.float32)
        p = jax.lax.dot_general(
            self32, xyzn, (((1,), (0,)), ((), ())),
            preferred_element_type=jnp.float32)
        nbx = jnp.where(slotm == t, jnp.broadcast_to(p[:, 0:1], (_G, _M)), nbx)
        nby = jnp.where(slotm == t, jnp.broadcast_to(p[:, 1:2], (_G, _M)), nby)
        nbz = jnp.where(slotm == t, jnp.broadcast_to(p[:, 2:3], (_G, _M)), nbz)
        return nbx, nby, nbz

    nb0 = jnp.zeros((_G, _M), jnp.float32)
    nbx, nby, nbz = jax.lax.fori_loop(0, _M, step, (nb0, nb0, nb0))
    nb_ref[0, 0] = nbx - cx
    nb_ref[0, 1] = nby - cy
    nb_ref[0, 2] = nbz - cz


def kernel(xyz):
    B, N, _ = xyz.shape
    xyz_t = jnp.transpose(xyz, (2, 0, 1))
    fps_idx, cen = pl.pallas_call(
        _fps_kernel,
        out_shape=(
            jax.ShapeDtypeStruct((B, _G), jnp.int32),
            jax.ShapeDtypeStruct((3, B, _G), jnp.float32),
        ),
        scratch_shapes=[pltpu.VMEM((B, N), jnp.float32)],
    )(xyz_t)

    center = jnp.transpose(cen, (1, 2, 0))
    xyz_b = jnp.transpose(xyz, (0, 2, 1))
    nb = pl.pallas_call(
        _knn_kernel,
        grid=(B,),
        in_specs=[
            pl.BlockSpec((1, 3, N), lambda b: (b, 0, 0)),
            pl.BlockSpec((1, N, 3), lambda b: (b, 0, 0)),
            pl.BlockSpec((1, _G, 3), lambda b: (b, 0, 0)),
        ],
        out_specs=pl.BlockSpec((1, 3, _G, _M), lambda b: (b, 0, 0, 0)),
        out_shape=jax.ShapeDtypeStruct((B, 3, _G, _M), jnp.float32),
        scratch_shapes=[pltpu.VMEM((_G, N), jnp.float32)],
        compiler_params=pltpu.CompilerParams(
            dimension_semantics=("parallel",)),
    )(xyz_b, xyz, center)

    neighborhood = jnp.transpose(nb, (0, 2, 3, 1))
    return neighborhood, center, fps_idx
